# Initial kernel scaffold; baseline (speedup 1.0000x reference)
import jax
import jax.numpy as jnp
from jax.experimental import pallas as pl


def kernel(state_embedding, urgency_vector, signal_summary, W_node, b_node, caste_table, Wq, bq, Wk, bk, Wv, bv, edge_bias_table, Wo, bo, gamma, beta):
    raise NotImplementedError("write your pallas kernel here")


# speedup vs baseline: 1.5765x; 1.5765x over previous
import functools
import math

import numpy as np
import jax
import jax.numpy as jnp
from jax.experimental import pallas as pl
from jax.experimental.pallas import tpu as pltpu

NUM_CASTES = 5
NUM_NODES = 11
EMB = 128
HID = 64
H = 2
D = HID // H

_NODE_INDEX = {'atp_executor': 0, 'order_tracking': 1, 'po_creation': 2, 'rebalancing': 3, 'subcontracting': 4, 'safety_stock': 5, 'forecast_adj': 6, 'quality': 7, 'maintenance': 8, 'mo_execution': 9, 'to_execution': 10}
_TO_CASTE = {'atp_executor': 0, 'order_tracking': 0, 'po_creation': 1, 'rebalancing': 1, 'subcontracting': 1, 'safety_stock': 2, 'forecast_adj': 2, 'quality': 3, 'maintenance': 3, 'mo_execution': 4, 'to_execution': 4}
_EDGES = [('atp_executor','po_creation'),('atp_executor','rebalancing'),('atp_executor','subcontracting'),('atp_executor','safety_stock'),('order_tracking','po_creation'),('po_creation','atp_executor'),('po_creation','order_tracking'),('rebalancing','atp_executor'),('rebalancing','to_execution'),('subcontracting','mo_execution'),('safety_stock','po_creation'),('safety_stock','atp_executor'),('forecast_adj','po_creation'),('forecast_adj','safety_stock'),('forecast_adj','atp_executor'),('quality','atp_executor'),('quality','mo_execution'),('maintenance','mo_execution'),('maintenance','subcontracting'),('mo_execution','atp_executor'),('mo_execution','po_creation'),('mo_execution','subcontracting'),('to_execution','order_tracking'),('to_execution','rebalancing')]

_SRC = np.array([_NODE_INDEX[s] for s, t in _EDGES], dtype=np.int32)
_TGT = np.array([_NODE_INDEX[t] for s, t in _EDGES], dtype=np.int32)
_ETYPE = np.array([_TO_CASTE[s] * NUM_CASTES + _TO_CASTE[t] for s, t in _EDGES], dtype=np.int32)
_CASTES = np.array([_TO_CASTE[n] for n in sorted(_NODE_INDEX, key=lambda k: _NODE_INDEX[k])], dtype=np.int32)
NE = len(_EDGES)

_PERM = np.argsort(_TGT, kind='stable')
_SRCP = _SRC[_PERM]
_TGTP = _TGT[_PERM]
_ETP = _ETYPE[_PERM]
_SEGS = []
_i = 0
while _i < NE:
    _j = _i
    while _j < NE and _TGTP[_j] == _TGTP[_i]:
        _j += 1
    _SEGS.append((int(_i), int(_j)))
    _i = _j
_HAS_IN = np.zeros(NUM_NODES, dtype=np.float32)
_HAS_IN[np.unique(_TGTP)] = 1.0

NSC = 2 * NE
NW = 2 * NSC
NOUT = NUM_NODES * HID


def _prep(W_node, b_node, caste_table, Wq, bq, Wk, bk, Wv, bv,
          edge_bias_table, Wo, bo, gamma, beta):
    f32 = jnp.float32
    W_state = W_node[:EMB].astype(f32)
    w_u = W_node[EMB].astype(f32)
    W_caste = W_node[EMB + 1:].astype(f32)
    cnode = caste_table[_CASTES] @ W_caste + b_node
    Qc = (cnode @ Wq + bq).reshape(NUM_NODES, H, D)
    Kc = (cnode @ Wk + bk).reshape(NUM_NODES, H, D)
    Vc = (cnode @ Wv + bv).reshape(NUM_NODES, H, D)
    qu = (w_u @ Wq).reshape(H, D)
    ku = (w_u @ Wk).reshape(H, D)
    vu = (w_u @ Wv).reshape(H, D)
    WQ = (W_state @ Wq).reshape(EMB, H, D)
    inv = 1.0 / math.sqrt(D)

    aW = jnp.einsum('khd,hd->kh', WQ, ku) * inv
    gW = (jnp.einsum('khd,shd->ksh', WQ, Kc) * inv).reshape(EMB, NUM_NODES * H)
    C1 = jnp.einsum('hd,hd->h', qu, ku) * inv
    C2 = jnp.einsum('hd,shd->sh', qu, Kc) * inv
    C3 = jnp.einsum('thd,hd->th', Qc, ku) * inv
    C5 = jnp.einsum('thd,shd->tsh', Qc, Kc) * inv
    C4 = C5[_TGTP, _SRCP, :] + edge_bias_table[_ETP]

    Woh = Wo.reshape(H, D, HID)
    P = jnp.einsum('hd,hdo->ho', vu, Woh)
    Rm = jnp.einsum('shd,hdo->sho', Vc, Woh)
    VbWo = W_state @ Wv @ Wo

    Wdense = jnp.concatenate(
        [W_state + _HAS_IN[t] * VbWo for t in range(NUM_NODES)], axis=1)
    Wcomb = jnp.concatenate([Wdense, gW, aW], axis=1)

    tgt_oh = np.zeros((NE, NUM_NODES), dtype=np.float32)
    tgt_oh[np.arange(NE), _TGTP] = 1.0
    tgt_oh = jnp.asarray(tgt_oh)
    S_w = jnp.concatenate([
        jnp.einsum('et,eo->eto', tgt_oh, Rm[_SRCP, h]).reshape(NE, NOUT)
        for h in range(H)], axis=0)
    S_w2 = jnp.concatenate([
        jnp.einsum('et,o->eto', tgt_oh, P[h]).reshape(NE, NOUT)
        for h in range(H)], axis=0)
    S = jnp.concatenate([S_w, S_w2], axis=0)

    UW = jnp.einsum('tu,o->tuo', jnp.eye(NUM_NODES, dtype=f32), w_u).reshape(NUM_NODES, NOUT)

    EUs = np.zeros((NUM_NODES, NSC), dtype=np.float32)
    EUt = np.zeros((NUM_NODES, NSC), dtype=np.float32)
    SelG = np.zeros((NUM_NODES * H, NSC), dtype=np.float32)
    for h in range(H):
        for e in range(NE):
            c = h * NE + e
            EUs[_SRCP[e], c] = 1.0
            EUt[_TGTP[e], c] = 1.0
            SelG[_SRCP[e] * H + h, c] = 1.0
    EUs, EUt, SelG = jnp.asarray(EUs), jnp.asarray(EUt), jnp.asarray(SelG)

    C3v = jnp.concatenate([C3[_TGTP, h] for h in range(H)])
    C2v = jnp.concatenate([C2[_SRCP, h] for h in range(H)])
    C4v = jnp.concatenate([C4[:, h] for h in range(H)])
    C1v = jnp.concatenate([jnp.full((NE,), C1[h]) for h in range(H)])
    CE = jnp.stack([C3v, C2v, C4v, C1v], axis=0)

    ccf = (cnode + bo).reshape(NOUT)
    gam = jnp.tile(gamma.astype(f32), NUM_NODES)
    bet = jnp.tile(beta.astype(f32), NUM_NODES)
    V3 = jnp.stack([ccf, gam, bet], axis=0)

    E = np.zeros((NOUT, NUM_NODES), dtype=np.float32)
    for t in range(NUM_NODES):
        E[t * HID:(t + 1) * HID, t] = 1.0
    E = jnp.asarray(E)
    ET = E.T

    return dict(Wcomb=Wcomb, S=S, UW=UW, EUs=EUs, EUt=EUt, SelG=SelG,
                CE=CE, V3=V3, E=E, ET=ET)


def _scores_and_weights(u, G, A, CE, EUs, EUt, SelG):
    Bb = u.shape[0]
    u_src = jnp.dot(u, EUs, preferred_element_type=jnp.float32)
    u_tgt = jnp.dot(u, EUt, preferred_element_type=jnp.float32)
    G48 = jnp.dot(G, SelG, preferred_element_type=jnp.float32)
    A48 = jnp.concatenate([jnp.broadcast_to(A[:, 0:1], (Bb, NE)),
                           jnp.broadcast_to(A[:, 1:2], (Bb, NE))], axis=1)
    s = u_src * (A48 + CE[3] * u_tgt + CE[0]) + G48 + CE[1] * u_tgt + CE[2]
    parts = []
    for h in range(H):
        off = h * NE
        for (a, b) in _SEGS:
            seg = s[:, off + a:off + b]
            m = jnp.max(seg, axis=1, keepdims=True)
            e = jnp.exp(seg - m)
            parts.append(e / jnp.sum(e, axis=1, keepdims=True))
    w48 = jnp.concatenate(parts, axis=1)
    return jnp.concatenate([w48, w48 * u_src], axis=1)


def _tc_body(se_ref, u_ref, Wcomb_ref, EUs_ref, EUt_ref, SelG_ref, CE_ref,
             S_ref, UW_ref, E_ref, ET_ref, V3_ref, out_ref):
    se = se_ref[...]
    u = u_ref[...]
    X = jnp.dot(se, Wcomb_ref[...], preferred_element_type=jnp.float32)
    dense = X[:, :NOUT]
    G = X[:, NOUT:NOUT + NUM_NODES * H]
    A = X[:, NOUT + NUM_NODES * H:]
    ww = _scores_and_weights(u, G, A, CE_ref[...], EUs_ref[...],
                             EUt_ref[...], SelG_ref[...])
    V3 = V3_ref[...]
    y = (dense
         + jnp.dot(ww, S_ref[...], preferred_element_type=jnp.float32)
         + jnp.dot(u, UW_ref[...], preferred_element_type=jnp.float32)
         + V3[0])
    E = E_ref[...]
    ET = ET_ref[...]
    mu = jnp.dot(y, E, preferred_element_type=jnp.float32) * (1.0 / HID)
    d0 = y - jnp.dot(mu, ET, preferred_element_type=jnp.float32)
    var = jnp.dot(d0 * d0, E, preferred_element_type=jnp.float32) * (1.0 / HID)
    varf = jnp.dot(var, ET, preferred_element_type=jnp.float32)
    out_ref[...] = d0 * jax.lax.rsqrt(varf + 1e-5) * V3[1] + V3[2]


def _pick_block(Bsz):
    for bb in (512, 256, 128, 64, 32, 16, 8):
        if Bsz % bb == 0:
            return bb
    return Bsz


def kernel(state_embedding, urgency_vector, signal_summary, W_node, b_node,
           caste_table, Wq, bq, Wk, bk, Wv, bv, edge_bias_table, Wo, bo,
           gamma, beta):
    del signal_summary
    Bsz = state_embedding.shape[0]
    ops = _prep(W_node, b_node, caste_table, Wq, bq, Wk, bk, Wv, bv,
                edge_bias_table, Wo, bo, gamma, beta)
    Bb = _pick_block(Bsz)
    grid = Bsz // Bb
    full = lambda shape: pl.BlockSpec(shape, lambda i: (0, 0))
    out = pl.pallas_call(
        _tc_body,
        grid=(grid,),
        in_specs=[
            pl.BlockSpec((Bb, EMB), lambda i: (i, 0)),
            pl.BlockSpec((Bb, NUM_NODES), lambda i: (i, 0)),
            full(ops['Wcomb'].shape),
            full(ops['EUs'].shape),
            full(ops['EUt'].shape),
            full(ops['SelG'].shape),
            full(ops['CE'].shape),
            full(ops['S'].shape),
            full(ops['UW'].shape),
            full(ops['E'].shape),
            full(ops['ET'].shape),
            full(ops['V3'].shape),
        ],
        out_specs=pl.BlockSpec((Bb, NOUT), lambda i: (i, 0)),
        out_shape=jax.ShapeDtypeStruct((Bsz, NOUT), jnp.float32),
    )(state_embedding.astype(jnp.float32), urgency_vector.astype(jnp.float32),
      ops['Wcomb'], ops['EUs'], ops['EUt'], ops['SelG'], ops['CE'],
      ops['S'], ops['UW'], ops['E'], ops['ET'], ops['V3'])
    return out.reshape(Bsz, NUM_NODES, HID)


# speedup vs baseline: 4.2564x; 2.6999x over previous
import functools
import math

import numpy as np
import jax
import jax.numpy as jnp
from jax.experimental import pallas as pl
from jax.experimental.pallas import tpu as pltpu

NUM_CASTES = 5
NUM_NODES = 11
EMB = 128
HID = 64
H = 2
D = HID // H

_NODE_INDEX = {'atp_executor': 0, 'order_tracking': 1, 'po_creation': 2, 'rebalancing': 3, 'subcontracting': 4, 'safety_stock': 5, 'forecast_adj': 6, 'quality': 7, 'maintenance': 8, 'mo_execution': 9, 'to_execution': 10}
_TO_CASTE = {'atp_executor': 0, 'order_tracking': 0, 'po_creation': 1, 'rebalancing': 1, 'subcontracting': 1, 'safety_stock': 2, 'forecast_adj': 2, 'quality': 3, 'maintenance': 3, 'mo_execution': 4, 'to_execution': 4}
_EDGES = [('atp_executor','po_creation'),('atp_executor','rebalancing'),('atp_executor','subcontracting'),('atp_executor','safety_stock'),('order_tracking','po_creation'),('po_creation','atp_executor'),('po_creation','order_tracking'),('rebalancing','atp_executor'),('rebalancing','to_execution'),('subcontracting','mo_execution'),('safety_stock','po_creation'),('safety_stock','atp_executor'),('forecast_adj','po_creation'),('forecast_adj','safety_stock'),('forecast_adj','atp_executor'),('quality','atp_executor'),('quality','mo_execution'),('maintenance','mo_execution'),('maintenance','subcontracting'),('mo_execution','atp_executor'),('mo_execution','po_creation'),('mo_execution','subcontracting'),('to_execution','order_tracking'),('to_execution','rebalancing')]

_SRC = np.array([_NODE_INDEX[s] for s, t in _EDGES], dtype=np.int32)
_TGT = np.array([_NODE_INDEX[t] for s, t in _EDGES], dtype=np.int32)
_ETYPE = np.array([_TO_CASTE[s] * NUM_CASTES + _TO_CASTE[t] for s, t in _EDGES], dtype=np.int32)
_CASTES = np.array([_TO_CASTE[n] for n in sorted(_NODE_INDEX, key=lambda k: _NODE_INDEX[k])], dtype=np.int32)
NE = len(_EDGES)

_PERM = np.argsort(_TGT, kind='stable')
_SRCP = _SRC[_PERM]
_TGTP = _TGT[_PERM]
_ETP = _ETYPE[_PERM]
_SEGS = []
_i = 0
while _i < NE:
    _j = _i
    while _j < NE and _TGTP[_j] == _TGTP[_i]:
        _j += 1
    _SEGS.append((int(_i), int(_j)))
    _i = _j
_HAS_IN = np.zeros(NUM_NODES, dtype=np.float32)
_HAS_IN[np.unique(_TGTP)] = 1.0

NSC = 2 * NE
NW = 2 * NSC
NOUT = NUM_NODES * HID


def _prep(W_node, b_node, caste_table, Wq, bq, Wk, bk, Wv, bv,
          edge_bias_table, Wo, bo, gamma, beta):
    f32 = jnp.float32
    W_state = W_node[:EMB].astype(f32)
    w_u = W_node[EMB].astype(f32)
    W_caste = W_node[EMB + 1:].astype(f32)
    cnode = caste_table[_CASTES] @ W_caste + b_node
    Qc = (cnode @ Wq + bq).reshape(NUM_NODES, H, D)
    Kc = (cnode @ Wk + bk).reshape(NUM_NODES, H, D)
    Vc = (cnode @ Wv + bv).reshape(NUM_NODES, H, D)
    qu = (w_u @ Wq).reshape(H, D)
    ku = (w_u @ Wk).reshape(H, D)
    vu = (w_u @ Wv).reshape(H, D)
    WQ = (W_state @ Wq).reshape(EMB, H, D)
    inv = 1.0 / math.sqrt(D)

    aW = jnp.einsum('khd,hd->kh', WQ, ku) * inv
    gW = (jnp.einsum('khd,shd->ksh', WQ, Kc) * inv).reshape(EMB, NUM_NODES * H)
    C1 = jnp.einsum('hd,hd->h', qu, ku) * inv
    C2 = jnp.einsum('hd,shd->sh', qu, Kc) * inv
    C3 = jnp.einsum('thd,hd->th', Qc, ku) * inv
    C5 = jnp.einsum('thd,shd->tsh', Qc, Kc) * inv
    C4 = C5[_TGTP, _SRCP, :] + edge_bias_table[_ETP]

    Woh = Wo.reshape(H, D, HID)
    P = jnp.einsum('hd,hdo->ho', vu, Woh)
    Rm = jnp.einsum('shd,hdo->sho', Vc, Woh)
    VbWo = W_state @ Wv @ Wo

    WT704 = jnp.concatenate(
        [W_state + _HAS_IN[t] * VbWo for t in range(NUM_NODES)], axis=1)
    WGA = jnp.concatenate([gW, aW], axis=1)

    tgt_oh = np.zeros((NE, NUM_NODES), dtype=np.float32)
    tgt_oh[np.arange(NE), _TGTP] = 1.0
    tgt_oh = jnp.asarray(tgt_oh)
    S_w = jnp.concatenate([
        jnp.einsum('et,eo->eto', tgt_oh, Rm[_SRCP, h]).reshape(NE, NOUT)
        for h in range(H)], axis=0)
    S_w2 = jnp.concatenate([
        jnp.einsum('et,o->eto', tgt_oh, P[h]).reshape(NE, NOUT)
        for h in range(H)], axis=0)

    UW = jnp.einsum('tu,o->tuo', jnp.eye(NUM_NODES, dtype=f32), w_u).reshape(NUM_NODES, NOUT)

    EUst = np.zeros((NUM_NODES, 2 * NSC), dtype=np.float32)
    MG = np.zeros((NUM_NODES * H + H, NSC), dtype=np.float32)
    MA = np.zeros((NUM_NODES * H + H, NSC), dtype=np.float32)
    Gmat = np.zeros((NSC, NSC), dtype=np.float32)
    for h in range(H):
        for e in range(NE):
            c = h * NE + e
            EUst[_SRCP[e], c] = 1.0
            EUst[_TGTP[e], NSC + c] = 1.0
            MG[_SRCP[e] * H + h, c] = 1.0
            MA[NUM_NODES * H + h, c] = 1.0
        for (a, b) in _SEGS:
            for e in range(a, b):
                for e2 in range(a, b):
                    Gmat[h * NE + e, h * NE + e2] = 1.0
    EUst, MG, MA, Gmat = map(jnp.asarray, (EUst, MG, MA, Gmat))

    C3v = jnp.concatenate([C3[_TGTP, h] for h in range(H)])
    C2v = jnp.concatenate([C2[_SRCP, h] for h in range(H)])
    C4v = jnp.concatenate([C4[:, h] for h in range(H)])
    C1v = jnp.concatenate([jnp.full((NE,), C1[h]) for h in range(H)])
    CE = jnp.stack([C3v, C2v, C4v, C1v], axis=0)

    ccf = (cnode + bo).reshape(NOUT)
    gam = jnp.tile(gamma.astype(f32), NUM_NODES)
    bet = jnp.tile(beta.astype(f32), NUM_NODES)
    V3 = jnp.stack([ccf, gam, bet], axis=0)

    E = np.zeros((NOUT, NUM_NODES), dtype=np.float32)
    for t in range(NUM_NODES):
        E[t * HID:(t + 1) * HID, t] = 1.0
    E = jnp.asarray(E)
    ET = E.T

    return dict(WT704=WT704, WGA=WGA, EUst=EUst, MG=MG, MA=MA, Gmat=Gmat,
                S_w=S_w, S_w2=S_w2, UW=UW, CE=CE, V3=V3, E=E, ET=ET)


def _dot(a, b):
    return jnp.dot(a, b, preferred_element_type=jnp.float32)


def _tc_body(se_ref, u_ref, WT704_ref, WGA_ref, EUst_ref, MG_ref, MA_ref,
             Gmat_ref, S_w_ref, S_w2_ref, UW_ref, CE_ref, V3_ref, E_ref,
             ET_ref, out_ref):
    se = se_ref[...]
    u = u_ref[...]
    CE = CE_ref[...]
    V3 = V3_ref[...]
    dense = _dot(se, WT704_ref[...])
    GA = _dot(se, WGA_ref[...])
    UU = _dot(u, EUst_ref[...])
    u_src = UU[:, :NSC]
    u_tgt = UU[:, NSC:]
    G48 = _dot(GA, MG_ref[...])
    A48 = _dot(GA, MA_ref[...])
    s = u_src * (A48 + CE[3] * u_tgt + CE[0]) + G48 + CE[1] * u_tgt + CE[2]
    m = jnp.max(s, axis=1, keepdims=True)
    ez = jnp.exp(s - m)
    denom = _dot(ez, Gmat_ref[...])
    w = ez / denom
    w2 = w * u_src
    y = (dense + _dot(w, S_w_ref[...]) + _dot(w2, S_w2_ref[...])
         + _dot(u, UW_ref[...]) + V3[0])
    E = E_ref[...]
    ET = ET_ref[...]
    mu = _dot(y, E) * (1.0 / HID)
    d0 = y - _dot(mu, ET)
    var = _dot(d0 * d0, E) * (1.0 / HID)
    qf = _dot(jax.lax.rsqrt(var + 1e-5), ET)
    out_ref[...] = d0 * qf * V3[1] + V3[2]


def _pick_block(Bsz):
    for bb in (2048, 1024, 512, 256, 128, 64, 32, 16, 8):
        if Bsz % bb == 0:
            return bb
    return Bsz


def kernel(state_embedding, urgency_vector, signal_summary, W_node, b_node,
           caste_table, Wq, bq, Wk, bk, Wv, bv, edge_bias_table, Wo, bo,
           gamma, beta):
    del signal_summary
    Bsz = state_embedding.shape[0]
    ops = _prep(W_node, b_node, caste_table, Wq, bq, Wk, bk, Wv, bv,
                edge_bias_table, Wo, bo, gamma, beta)
    Bb = _pick_block(Bsz)
    grid = Bsz // Bb
    names = ('WT704', 'WGA', 'EUst', 'MG', 'MA', 'Gmat', 'S_w', 'S_w2',
             'UW', 'CE', 'V3', 'E', 'ET')
    full = lambda shape: pl.BlockSpec(shape, lambda i: (0, 0))
    out = pl.pallas_call(
        _tc_body,
        grid=(grid,),
        in_specs=[
            pl.BlockSpec((Bb, EMB), lambda i: (i, 0)),
            pl.BlockSpec((Bb, NUM_NODES), lambda i: (i, 0)),
        ] + [full(ops[n].shape) for n in names],
        out_specs=pl.BlockSpec((Bb, NOUT), lambda i: (i, 0)),
        out_shape=jax.ShapeDtypeStruct((Bsz, NOUT), jnp.float32),
    )(state_embedding.astype(jnp.float32), urgency_vector.astype(jnp.float32),
      *[ops[n] for n in names])
    return out.reshape(Bsz, NUM_NODES, HID)


# speedup vs baseline: 4.6647x; 1.0959x over previous
import functools
import math

import numpy as np
import jax
import jax.numpy as jnp
from jax.experimental import pallas as pl
from jax.experimental.pallas import tpu as pltpu

NUM_CASTES = 5
NUM_NODES = 11
EMB = 128
HID = 64
H = 2
D = HID // H

_NODE_INDEX = {'atp_executor': 0, 'order_tracking': 1, 'po_creation': 2, 'rebalancing': 3, 'subcontracting': 4, 'safety_stock': 5, 'forecast_adj': 6, 'quality': 7, 'maintenance': 8, 'mo_execution': 9, 'to_execution': 10}
_TO_CASTE = {'atp_executor': 0, 'order_tracking': 0, 'po_creation': 1, 'rebalancing': 1, 'subcontracting': 1, 'safety_stock': 2, 'forecast_adj': 2, 'quality': 3, 'maintenance': 3, 'mo_execution': 4, 'to_execution': 4}
_EDGES = [('atp_executor','po_creation'),('atp_executor','rebalancing'),('atp_executor','subcontracting'),('atp_executor','safety_stock'),('order_tracking','po_creation'),('po_creation','atp_executor'),('po_creation','order_tracking'),('rebalancing','atp_executor'),('rebalancing','to_execution'),('subcontracting','mo_execution'),('safety_stock','po_creation'),('safety_stock','atp_executor'),('forecast_adj','po_creation'),('forecast_adj','safety_stock'),('forecast_adj','atp_executor'),('quality','atp_executor'),('quality','mo_execution'),('maintenance','mo_execution'),('maintenance','subcontracting'),('mo_execution','atp_executor'),('mo_execution','po_creation'),('mo_execution','subcontracting'),('to_execution','order_tracking'),('to_execution','rebalancing')]

_SRC = np.array([_NODE_INDEX[s] for s, t in _EDGES], dtype=np.int32)
_TGT = np.array([_NODE_INDEX[t] for s, t in _EDGES], dtype=np.int32)
_ETYPE = np.array([_TO_CASTE[s] * NUM_CASTES + _TO_CASTE[t] for s, t in _EDGES], dtype=np.int32)
_CASTES = np.array([_TO_CASTE[n] for n in sorted(_NODE_INDEX, key=lambda k: _NODE_INDEX[k])], dtype=np.int32)
NE = len(_EDGES)

_PERM = np.argsort(_TGT, kind='stable')
_SRCP = _SRC[_PERM]
_TGTP = _TGT[_PERM]
_ETP = _ETYPE[_PERM]
_SEGS = []
_i = 0
while _i < NE:
    _j = _i
    while _j < NE and _TGTP[_j] == _TGTP[_i]:
        _j += 1
    _SEGS.append((int(_i), int(_j)))
    _i = _j
_HAS_IN = np.zeros(NUM_NODES, dtype=np.float32)
_HAS_IN[np.unique(_TGTP)] = 1.0

NSC = 2 * NE
NW = 2 * NSC
NOUT = NUM_NODES * HID


def _prep(W_node, b_node, caste_table, Wq, bq, Wk, bk, Wv, bv,
          edge_bias_table, Wo, bo, gamma, beta):
    f32 = jnp.float32
    W_state = W_node[:EMB].astype(f32)
    w_u = W_node[EMB].astype(f32)
    W_caste = W_node[EMB + 1:].astype(f32)
    cnode = caste_table[_CASTES] @ W_caste + b_node
    Qc = (cnode @ Wq + bq).reshape(NUM_NODES, H, D)
    Kc = (cnode @ Wk + bk).reshape(NUM_NODES, H, D)
    Vc = (cnode @ Wv + bv).reshape(NUM_NODES, H, D)
    qu = (w_u @ Wq).reshape(H, D)
    ku = (w_u @ Wk).reshape(H, D)
    vu = (w_u @ Wv).reshape(H, D)
    WQ = (W_state @ Wq).reshape(EMB, H, D)
    inv = 1.0 / math.sqrt(D)

    aW = jnp.einsum('khd,hd->kh', WQ, ku) * inv
    gW = (jnp.einsum('khd,shd->ksh', WQ, Kc) * inv).reshape(EMB, NUM_NODES * H)
    C1 = jnp.einsum('hd,hd->h', qu, ku) * inv
    C2 = jnp.einsum('hd,shd->sh', qu, Kc) * inv
    C3 = jnp.einsum('thd,hd->th', Qc, ku) * inv
    C5 = jnp.einsum('thd,shd->tsh', Qc, Kc) * inv
    C4 = C5[_TGTP, _SRCP, :] + edge_bias_table[_ETP]

    Woh = Wo.reshape(H, D, HID)
    P = jnp.einsum('hd,hdo->ho', vu, Woh)
    Rm = jnp.einsum('shd,hdo->sho', Vc, Woh)
    VbWo = W_state @ Wv @ Wo

    WT704 = jnp.concatenate(
        [W_state + _HAS_IN[t] * VbWo for t in range(NUM_NODES)], axis=1)
    WGA = jnp.concatenate([gW, aW], axis=1)

    tgt_oh = np.zeros((NE, NUM_NODES), dtype=np.float32)
    tgt_oh[np.arange(NE), _TGTP] = 1.0
    tgt_oh = jnp.asarray(tgt_oh)
    S_w = jnp.concatenate([
        jnp.einsum('et,eo->eto', tgt_oh, Rm[_SRCP, h]).reshape(NE, NOUT)
        for h in range(H)], axis=0)
    S_w2 = jnp.concatenate([
        jnp.einsum('et,o->eto', tgt_oh, P[h]).reshape(NE, NOUT)
        for h in range(H)], axis=0)

    UW = jnp.einsum('tu,o->tuo', jnp.eye(NUM_NODES, dtype=f32), w_u).reshape(NUM_NODES, NOUT)

    EUst = np.zeros((NUM_NODES, 2 * NSC), dtype=np.float32)
    MG = np.zeros((NUM_NODES * H + H, NSC), dtype=np.float32)
    MA = np.zeros((NUM_NODES * H + H, NSC), dtype=np.float32)
    Gmat = np.zeros((NSC, NSC), dtype=np.float32)
    for h in range(H):
        for e in range(NE):
            c = h * NE + e
            EUst[_SRCP[e], c] = 1.0
            EUst[_TGTP[e], NSC + c] = 1.0
            MG[_SRCP[e] * H + h, c] = 1.0
            MA[NUM_NODES * H + h, c] = 1.0
        for (a, b) in _SEGS:
            for e in range(a, b):
                for e2 in range(a, b):
                    Gmat[h * NE + e, h * NE + e2] = 1.0
    EUst, MG, MA, Gmat = map(jnp.asarray, (EUst, MG, MA, Gmat))

    C3v = jnp.concatenate([C3[_TGTP, h] for h in range(H)])
    C2v = jnp.concatenate([C2[_SRCP, h] for h in range(H)])
    C4v = jnp.concatenate([C4[:, h] for h in range(H)])
    C1v = jnp.concatenate([jnp.full((NE,), C1[h]) for h in range(H)])
    CE = jnp.stack([C3v, C2v, C4v, C1v], axis=0)

    ccf = (cnode + bo).reshape(NOUT)
    gam = jnp.tile(gamma.astype(f32), NUM_NODES)
    bet = jnp.tile(beta.astype(f32), NUM_NODES)
    V3 = jnp.stack([ccf, gam, bet], axis=0)

    E = np.zeros((NOUT, NUM_NODES), dtype=np.float32)
    for t in range(NUM_NODES):
        E[t * HID:(t + 1) * HID, t] = 1.0
    E = jnp.asarray(E)
    ET = E.T

    bf = jnp.bfloat16
    W1 = jnp.concatenate([WT704, WT704 @ E * (1.0 / HID)], axis=1)
    Sb = jnp.concatenate([S_w, S_w2, UW], axis=0)
    Sbig = jnp.concatenate([Sb, Sb @ E * (1.0 / HID)], axis=1)
    MUC = ((ccf @ E) * (1.0 / HID)).reshape(1, NUM_NODES)
    return dict(W1=W1.astype(bf), WGA=WGA.astype(bf), EUst=EUst.astype(bf),
                MG=MG.astype(bf), MA=MA.astype(bf), Gmat=Gmat.astype(bf),
                Sbig=Sbig.astype(bf), CE=CE, MUC=MUC, V3=V3,
                E=E.astype(bf), ET=ET.astype(bf))


def _dot(a, b):
    return jnp.dot(a.astype(jnp.bfloat16), b,
                   preferred_element_type=jnp.float32)


def _tc_body(se_ref, u_ref, W1_ref, WGA_ref, EUst_ref, MG_ref, MA_ref,
             Gmat_ref, Sbig_ref, CE_ref, MUC_ref, V3_ref, E_ref,
             ET_ref, out_ref):
    se = se_ref[...]
    u = u_ref[...]
    CE = CE_ref[...]
    V3 = V3_ref[...]
    X = _dot(se, W1_ref[...])
    GA = _dot(se, WGA_ref[...])
    UU = _dot(u, EUst_ref[...])
    u_src = UU[:, :NSC]
    u_tgt = UU[:, NSC:]
    G48 = _dot(GA, MG_ref[...])
    A48 = _dot(GA, MA_ref[...])
    s = u_src * (A48 + CE[3] * u_tgt + CE[0]) + G48 + CE[1] * u_tgt + CE[2]
    m = jnp.max(s, axis=1, keepdims=True)
    ez = jnp.exp(s - m)
    denom = _dot(ez, Gmat_ref[...])
    w = ez / denom
    cat = jnp.concatenate([w, w * u_src, u], axis=1)
    Y = _dot(cat, Sbig_ref[...])
    y = X[:, :NOUT] + Y[:, :NOUT] + V3[0]
    mu = X[:, NOUT:] + Y[:, NOUT:] + MUC_ref[...][0]
    var = _dot(y * y, E_ref[...]) * (1.0 / HID) - mu * mu
    q = jax.lax.rsqrt(var + 1e-5)
    qf = _dot(q, ET_ref[...])
    muf = _dot(mu * q, ET_ref[...])
    out_ref[...] = (y * qf - muf) * V3[1] + V3[2]


def _pick_block(Bsz):
    for bb in (2048, 1024, 512, 256, 128, 64, 32, 16, 8):
        if Bsz % bb == 0:
            return bb
    return Bsz


def kernel(state_embedding, urgency_vector, signal_summary, W_node, b_node,
           caste_table, Wq, bq, Wk, bk, Wv, bv, edge_bias_table, Wo, bo,
           gamma, beta):
    del signal_summary
    Bsz = state_embedding.shape[0]
    ops = _prep(W_node, b_node, caste_table, Wq, bq, Wk, bk, Wv, bv,
                edge_bias_table, Wo, bo, gamma, beta)
    Bb = _pick_block(Bsz)
    grid = Bsz // Bb
    names = ('W1', 'WGA', 'EUst', 'MG', 'MA', 'Gmat', 'Sbig',
             'CE', 'MUC', 'V3', 'E', 'ET')
    full = lambda shape: pl.BlockSpec(shape, lambda i: (0, 0))
    out = pl.pallas_call(
        _tc_body,
        grid=(grid,),
        in_specs=[
            pl.BlockSpec((Bb, EMB), lambda i: (i, 0)),
            pl.BlockSpec((Bb, NUM_NODES), lambda i: (i, 0)),
        ] + [full(ops[n].shape) for n in names],
        out_specs=pl.BlockSpec((Bb, NOUT), lambda i: (i, 0)),
        out_shape=jax.ShapeDtypeStruct((Bsz, NOUT), jnp.float32),
    )(state_embedding.astype(jnp.float32), urgency_vector.astype(jnp.float32),
      *[ops[n] for n in names])
    return out.reshape(Bsz, NUM_NODES, HID)
